# baseline (speedup 1.0000x reference)
import functools

import jax
import jax.numpy as jnp
from jax import lax
from jax.experimental import pallas as pl
from jax.experimental.pallas import tpu as pltpu
from jax.experimental.pallas import tpu_sc as plsc

N_SAMPLES = 16384
N_CLASSES = 1000
LANES = 16
N_WORKERS = 32
SPW = N_SAMPLES // N_WORKERS
SUBS = SPW // LANES
HALF = SUBS // 2
CHUNK = 40
N_CHUNKS = N_CLASSES // CHUNK
N_SLOTS = 4
NEG = -10.0 ** 10
MARGIN_FY = 0.8
MARGIN_FNY = 0.3


def _sc_body(xt_hbm, lbl_hbm, out_hbm, xbuf, lblbuf, mbuf, fybuf, ostage,
             *sems):
    wid = lax.axis_index("c") * 16 + lax.axis_index("s")
    s0 = wid * SPW

    pltpu.sync_copy(lbl_hbm.at[pl.ds(s0, SPW)], lblbuf)

    lane = lax.iota(jnp.int32, LANES)
    neg_vec = jnp.full((LANES,), NEG, jnp.float32)
    zero_vec = jnp.zeros((LANES,), jnp.float32)
    for sub in range(SUBS):
        mbuf[pl.ds(sub * LANES, LANES)] = neg_vec
        fybuf[pl.ds(sub * LANES, LANES)] = zero_vec

    def dma(k, slot, sem):
        src = xt_hbm.at[pl.ds(CHUNK * k, CHUNK), pl.ds(s0, SPW)]
        dst = xbuf.at[slot, :, :]
        return pltpu.make_async_copy(src, dst, sem)

    for k in range(N_SLOTS - 1):
        dma(k, k, sems[k]).start()

    def chunk_body(k, carry):
        slot = lax.rem(k, N_SLOTS)
        base_vec = jnp.full((LANES,), CHUNK, jnp.int32) * k
        slot_vec = jnp.full((LANES,), 1, jnp.int32) * slot

        @pl.when(slot == 0)
        def _():
            dma(k, 0, sems[0]).wait()

        @pl.when(slot == 1)
        def _():
            dma(k, 1, sems[1]).wait()

        @pl.when(slot == 2)
        def _():
            dma(k, 2, sems[2]).wait()

        @pl.when(slot == 3)
        def _():
            dma(k, 3, sems[3]).wait()

        pre = k + N_SLOTS - 1

        @pl.when(pre < N_CHUNKS)
        def _():
            pslot = lax.rem(pre, N_SLOTS)

            @pl.when(pslot == 0)
            def _():
                dma(pre, 0, sems[0]).start()

            @pl.when(pslot == 1)
            def _():
                dma(pre, 1, sems[1]).start()

            @pl.when(pslot == 2)
            def _():
                dma(pre, 2, sems[2]).start()

            @pl.when(pslot == 3)
            def _():
                dma(pre, 3, sems[3]).start()

        for sub in range(SUBS):
            lbl_sub = lblbuf[pl.ds(sub * LANES, LANES)]
            rel = lbl_sub - base_vec
            mask = (rel >= 0) & (rel < CHUNK)
            relc = jnp.clip(rel, 0, CHUNK - 1)
            cvec = sub * LANES + lane
            got = plsc.load_gather(xbuf, [slot_vec, relc, cvec], mask=mask)
            fyo = fybuf[pl.ds(sub * LANES, LANES)]
            fybuf[pl.ds(sub * LANES, LANES)] = jnp.where(mask, got, fyo)
            plsc.store_scatter(xbuf, [slot_vec, relc, cvec], neg_vec,
                               mask=mask)

        for h in range(2):
            ms = tuple(mbuf[pl.ds((h * HALF + i) * LANES, LANES)]
                       for i in range(HALF))

            def row_body(ri, ms, _h=h):
                r = 2 * ri
                ms = tuple(
                    jnp.maximum(
                        ms[i],
                        xbuf[slot, r, pl.ds((_h * HALF + i) * LANES, LANES)])
                    for i in range(HALF))
                return tuple(
                    jnp.maximum(
                        ms[i],
                        xbuf[slot, r + 1,
                             pl.ds((_h * HALF + i) * LANES, LANES)])
                    for i in range(HALF))

            ms = lax.fori_loop(0, CHUNK // 2, row_body, ms)
            for i in range(HALF):
                mbuf[pl.ds((h * HALF + i) * LANES, LANES)] = ms[i]

        return carry

    lax.fori_loop(0, N_CHUNKS, chunk_body, 0)

    acc = jnp.zeros((LANES,), jnp.float32)
    for sub in range(SUBS):
        fy = fybuf[pl.ds(sub * LANES, LANES)]
        m = mbuf[pl.ds(sub * LANES, LANES)]
        acc = acc + (jnp.maximum(MARGIN_FY - fy, 0.0)
                     + jnp.maximum(MARGIN_FNY + m, 0.0))
    ostage[...] = acc * (1.0 / N_SAMPLES)
    pltpu.sync_copy(ostage, out_hbm.at[wid])


_sc_loss = functools.partial(
    pl.kernel,
    out_type=jax.ShapeDtypeStruct((N_WORKERS, LANES), jnp.float32),
    mesh=plsc.VectorSubcoreMesh(core_axis_name="c", subcore_axis_name="s"),
    compiler_params=pltpu.CompilerParams(needs_layout_passes=False,
                                         use_tc_tiling_on_sc=True),
    scratch_types=[
        pltpu.VMEM((N_SLOTS, CHUNK, SPW), jnp.float32),
        pltpu.VMEM((SPW,), jnp.int32),
        pltpu.VMEM((SPW,), jnp.float32),
        pltpu.VMEM((SPW,), jnp.float32),
        pltpu.VMEM((LANES,), jnp.float32),
    ] + [pltpu.SemaphoreType.DMA] * N_SLOTS,
)(_sc_body)


def kernel(x, label):
    parts = _sc_loss(x.T, label.astype(jnp.int32))
    return jnp.sum(parts)


# speedup vs baseline: 1.2742x; 1.2742x over previous
import functools

import jax
import jax.numpy as jnp
from jax import lax
from jax.experimental import pallas as pl
from jax.experimental.pallas import tpu as pltpu
from jax.experimental.pallas import tpu_sc as plsc

N_SAMPLES = 16384
N_CLASSES = 1000
LANES = 16
N_WORKERS = 32

SC_N = 8192
TC_N = N_SAMPLES - SC_N

SPW = SC_N // N_WORKERS
SB = 128
N_BLOCKS = SPW // SB
SUBS = SB // LANES
CHUNK = 128
ROWS = [CHUNK] * 7 + [N_CLASSES - 7 * CHUNK]
N_CHUNKS = len(ROWS)
N_SLOTS = 4

W_TC = 512
NEG = -10.0 ** 10
MARGIN_FY = 0.8
MARGIN_FNY = 0.3


def _sc_body(xt_hbm, lbl_hbm, out_hbm, xbuf, lblbuf, ostage, *sems):
    wid = lax.axis_index("c") * 16 + lax.axis_index("s")
    s0 = TC_N + wid * SPW

    pltpu.sync_copy(lbl_hbm.at[pl.ds(s0, SPW)], lblbuf)

    lane = lax.iota(jnp.int32, LANES)

    def dma(b, k, sem):
        slot = k % N_SLOTS
        src = xt_hbm.at[pl.ds(CHUNK * k, ROWS[k]), pl.ds(s0 + b * SB, SB)]
        dst = xbuf.at[slot, pl.ds(0, ROWS[k]), :]
        return pltpu.make_async_copy(src, dst, sem)

    for k in range(N_SLOTS - 1):
        dma(0, k, sems[k % N_SLOTS]).start()

    def block_body(b, acc):
        m = [jnp.full((LANES,), NEG, jnp.float32) for _ in range(SUBS)]
        fy = [jnp.zeros((LANES,), jnp.float32) for _ in range(SUBS)]

        for k in range(N_CHUNKS):
            rows_k = ROWS[k]
            slot = k % N_SLOTS
            sem = sems[slot]
            dma(b, k, sem).wait()

            pre = k + N_SLOTS - 1
            if pre < N_CHUNKS:
                dma(b, pre, sems[pre % N_SLOTS]).start()
            else:
                pk = pre - N_CHUNKS

                @pl.when(b + 1 < N_BLOCKS)
                def _():
                    dma(b + 1, pk, sems[pk % N_SLOTS]).start()

            slot_vec = jnp.full((LANES,), slot, jnp.int32)
            neg_vec = jnp.full((LANES,), NEG, jnp.float32)
            for sub in range(SUBS):
                lbl_sub = lblbuf[pl.ds(b * SB + sub * LANES, LANES)]
                rel = lbl_sub - CHUNK * k
                mask = (rel >= 0) & (rel < rows_k)
                relc = jnp.clip(rel, 0, rows_k - 1)
                got = plsc.load_gather(
                    xbuf, [slot_vec, relc, sub * LANES + lane], mask=mask)
                fy[sub] = jnp.where(mask, got, fy[sub])
                plsc.store_scatter(
                    xbuf, [slot_vec, relc, sub * LANES + lane], neg_vec,
                    mask=mask)

            def row_body(ri, ms, _slot=slot):
                r = 4 * ri
                for dr in range(4):
                    ms = tuple(
                        jnp.maximum(
                            ms[i],
                            xbuf[_slot, r + dr, pl.ds(LANES * i, LANES)])
                        for i in range(SUBS))
                return ms

            m = list(lax.fori_loop(0, rows_k // 4, row_body, tuple(m)))

        for sub in range(SUBS):
            acc = acc + (jnp.maximum(MARGIN_FY - fy[sub], 0.0)
                         + jnp.maximum(MARGIN_FNY + m[sub], 0.0))
        return acc

    acc = lax.fori_loop(0, N_BLOCKS, block_body,
                        jnp.zeros((LANES,), jnp.float32))
    ostage[...] = acc * (1.0 / N_SAMPLES)
    pltpu.sync_copy(ostage, out_hbm.at[wid])


_sc_loss = functools.partial(
    pl.kernel,
    out_type=jax.ShapeDtypeStruct((N_WORKERS, LANES), jnp.float32),
    mesh=plsc.VectorSubcoreMesh(core_axis_name="c", subcore_axis_name="s"),
    compiler_params=pltpu.CompilerParams(needs_layout_passes=False,
                                         use_tc_tiling_on_sc=True),
    scratch_types=[
        pltpu.VMEM((N_SLOTS, CHUNK, SB), jnp.float32),
        pltpu.VMEM((SPW,), jnp.int32),
        pltpu.VMEM((LANES,), jnp.float32),
    ] + [pltpu.SemaphoreType.DMA] * N_SLOTS,
)(_sc_body)


def _tc_body(lbl_ref, xt_ref, out_ref):
    j = pl.program_id(0)
    lbl = lbl_ref[0, 0, :]
    xb = xt_ref[...]
    rows = lax.broadcasted_iota(jnp.int32, (N_CLASSES, W_TC), 0)
    eq = rows == lbl[None, :]
    fy = jnp.max(jnp.where(eq, xb, NEG), axis=0)
    m = jnp.max(jnp.where(eq, NEG, xb), axis=0)
    l = (jnp.maximum(MARGIN_FY - fy, 0.0)
         + jnp.maximum(MARGIN_FNY + m, 0.0))
    s = (jnp.sum(l) * (1.0 / N_SAMPLES)).reshape(1, 1)

    @pl.when(j == 0)
    def _():
        out_ref[...] = jnp.zeros((1, 1), jnp.float32)

    out_ref[...] += s


_tc_loss = pl.pallas_call(
    _tc_body,
    grid=(TC_N // W_TC,),
    in_specs=[
        pl.BlockSpec((1, 1, W_TC), lambda j: (j, 0, 0)),
        pl.BlockSpec((N_CLASSES, W_TC), lambda j: (0, j)),
    ],
    out_specs=pl.BlockSpec((1, 1), lambda j: (0, 0)),
    out_shape=jax.ShapeDtypeStruct((1, 1), jnp.float32),
)


def kernel(x, label):
    xt = x.T
    lbl = label.astype(jnp.int32)
    sc_parts = _sc_loss(xt, lbl)
    lbl3d = lbl.reshape(N_SAMPLES // W_TC, 1, W_TC)
    tc_part = _tc_loss(lbl3d, xt)
    return jnp.sum(sc_parts) + tc_part[0, 0]


# speedup vs baseline: 1.3158x; 1.0326x over previous
import functools

import jax
import jax.numpy as jnp
from jax import lax
from jax.experimental import pallas as pl
from jax.experimental.pallas import tpu as pltpu
from jax.experimental.pallas import tpu_sc as plsc

N_SAMPLES = 16384
N_CLASSES = 1000
LANES = 16
N_WORKERS = 32

SC_N = 8192
TC_N = N_SAMPLES - SC_N

SPW = SC_N // N_WORKERS
SB = 128
N_BLOCKS = SPW // SB
SUBS = SB // LANES
CHUNK = 128
ROWS = [CHUNK] * 7 + [N_CLASSES - 7 * CHUNK]
N_CHUNKS = len(ROWS)
N_SLOTS = 4

W_TC = 1024
NEG = -10.0 ** 10
MARGIN_FY = 0.8
MARGIN_FNY = 0.3


def _sc_body(xt_hbm, lbl_hbm, out_hbm, xbuf, lblbuf, ostage, *sems):
    wid = lax.axis_index("c") * 16 + lax.axis_index("s")
    s0 = TC_N + wid * SPW

    pltpu.sync_copy(lbl_hbm.at[pl.ds(s0, SPW)], lblbuf)

    lane = lax.iota(jnp.int32, LANES)

    def dma(b, k, sem):
        slot = k % N_SLOTS
        src = xt_hbm.at[pl.ds(CHUNK * k, ROWS[k]), pl.ds(s0 + b * SB, SB)]
        dst = xbuf.at[slot, pl.ds(0, ROWS[k]), :]
        return pltpu.make_async_copy(src, dst, sem)

    for k in range(N_SLOTS - 1):
        dma(0, k, sems[k % N_SLOTS]).start()

    def block_body(b, acc):
        m = [jnp.full((LANES,), NEG, jnp.float32) for _ in range(SUBS)]
        fy = [jnp.zeros((LANES,), jnp.float32) for _ in range(SUBS)]

        for k in range(N_CHUNKS):
            rows_k = ROWS[k]
            slot = k % N_SLOTS
            sem = sems[slot]
            dma(b, k, sem).wait()

            pre = k + N_SLOTS - 1
            if pre < N_CHUNKS:
                dma(b, pre, sems[pre % N_SLOTS]).start()
            else:
                pk = pre - N_CHUNKS

                @pl.when(b + 1 < N_BLOCKS)
                def _():
                    dma(b + 1, pk, sems[pk % N_SLOTS]).start()

            slot_vec = jnp.full((LANES,), slot, jnp.int32)
            neg_vec = jnp.full((LANES,), NEG, jnp.float32)
            for sub in range(SUBS):
                lbl_sub = lblbuf[pl.ds(b * SB + sub * LANES, LANES)]
                rel = lbl_sub - CHUNK * k
                mask = (rel >= 0) & (rel < rows_k)
                relc = jnp.clip(rel, 0, rows_k - 1)
                got = plsc.load_gather(
                    xbuf, [slot_vec, relc, sub * LANES + lane], mask=mask)
                fy[sub] = jnp.where(mask, got, fy[sub])
                plsc.store_scatter(
                    xbuf, [slot_vec, relc, sub * LANES + lane], neg_vec,
                    mask=mask)

            def row_body(ri, ms, _slot=slot):
                r = 4 * ri
                for dr in range(4):
                    ms = tuple(
                        jnp.maximum(
                            ms[i],
                            xbuf[_slot, r + dr, pl.ds(LANES * i, LANES)])
                        for i in range(SUBS))
                return ms

            m = list(lax.fori_loop(0, rows_k // 4, row_body, tuple(m)))

        for sub in range(SUBS):
            acc = acc + (jnp.maximum(MARGIN_FY - fy[sub], 0.0)
                         + jnp.maximum(MARGIN_FNY + m[sub], 0.0))
        return acc

    acc = lax.fori_loop(0, N_BLOCKS, block_body,
                        jnp.zeros((LANES,), jnp.float32))
    ostage[...] = acc * (1.0 / N_SAMPLES)
    pltpu.sync_copy(ostage, out_hbm.at[wid])


_sc_loss = functools.partial(
    pl.kernel,
    out_type=jax.ShapeDtypeStruct((N_WORKERS, LANES), jnp.float32),
    mesh=plsc.VectorSubcoreMesh(core_axis_name="c", subcore_axis_name="s"),
    compiler_params=pltpu.CompilerParams(needs_layout_passes=False,
                                         use_tc_tiling_on_sc=True),
    scratch_types=[
        pltpu.VMEM((N_SLOTS, CHUNK, SB), jnp.float32),
        pltpu.VMEM((SPW,), jnp.int32),
        pltpu.VMEM((LANES,), jnp.float32),
    ] + [pltpu.SemaphoreType.DMA] * N_SLOTS,
)(_sc_body)


def _tc_body(lbl_ref, xt_ref, out_ref):
    j = pl.program_id(0)
    lbl = lbl_ref[0, 0, :]
    xb = xt_ref[...]
    rows = lax.broadcasted_iota(jnp.int32, (N_CLASSES, W_TC), 0)
    eq = rows == lbl[None, :]
    fy = jnp.max(jnp.where(eq, xb, NEG), axis=0)
    m = jnp.max(jnp.where(eq, NEG, xb), axis=0)
    l = (jnp.maximum(MARGIN_FY - fy, 0.0)
         + jnp.maximum(MARGIN_FNY + m, 0.0))
    s = (jnp.sum(l) * (1.0 / N_SAMPLES)).reshape(1, 1)

    @pl.when(j == 0)
    def _():
        out_ref[...] = jnp.zeros((1, 1), jnp.float32)

    out_ref[...] += s


_tc_loss = pl.pallas_call(
    _tc_body,
    grid=(TC_N // W_TC,),
    in_specs=[
        pl.BlockSpec((1, 1, W_TC), lambda j: (j, 0, 0)),
        pl.BlockSpec((N_CLASSES, W_TC), lambda j: (0, j)),
    ],
    out_specs=pl.BlockSpec((1, 1), lambda j: (0, 0)),
    out_shape=jax.ShapeDtypeStruct((1, 1), jnp.float32),
)


def kernel(x, label):
    xt = x.T
    lbl = label.astype(jnp.int32)
    sc_parts = _sc_loss(xt, lbl)
    lbl3d = lbl.reshape(N_SAMPLES // W_TC, 1, W_TC)
    tc_part = _tc_loss(lbl3d, xt)
    return jnp.sum(sc_parts) + tc_part[0, 0]
